# Initial kernel scaffold; baseline (speedup 1.0000x reference)
import jax
import jax.numpy as jnp
from jax.experimental import pallas as pl


def kernel(input_images, patches, distortion, params):
    raise NotImplementedError("write your pallas kernel here")


# speedup vs baseline: 3.9767x; 3.9767x over previous
import functools

import jax
import jax.numpy as jnp
from jax import lax
from jax.experimental import pallas as pl
from jax.experimental.pallas import tpu as pltpu
from jax.experimental.pallas import tpu_sc as plsc

IMAGE_SIZE = 64
EMBED = 128
NUM_PATCHES = 64
SEQ = NUM_PATCHES + 1
NUM_HEADS = 8
HEAD_DIM = EMBED // NUM_HEADS
HIDDEN = 256
NUM_BLOCKS = 2
BATCH = 512
MIN_ANGLE = -3.323
MAX_ANGLE = 3.232
PIX = IMAGE_SIZE * IMAGE_SIZE

SC_CORES = 2
SC_SUBCORES = 16
IMGS_PER_WORKER = BATCH // (SC_CORES * SC_SUBCORES)


BB = 16
SEQP = 128


def _ln(x, g, b):
    mu = jnp.mean(x, axis=-1, keepdims=True)
    var = jnp.mean((x - mu) ** 2, axis=-1, keepdims=True)
    return (x - mu) / jnp.sqrt(var + 1e-5) * g + b


def _transformer_body(x_ref, ln1g, ln1b, wqkv, bqkv, wo, bo, temp,
                      ln2g, ln2b, w1, b1, w2, b2, out_ref):
    diag = (lax.broadcasted_iota(jnp.int32, (SEQ, SEQ), 0)
            == lax.broadcasted_iota(jnp.int32, (SEQ, SEQ), 1))

    def per_image(i, _):
        x = x_ref[i][:SEQ]
        for blk in range(NUM_BLOCKS):
            h = _ln(x, ln1g[blk], ln1b[blk])
            qkv = jnp.dot(h, wqkv[blk],
                          preferred_element_type=jnp.float32) + bqkv[blk]
            t = temp[blk, 0]
            o_heads = []
            for hh in range(NUM_HEADS):
                q = qkv[:, hh * HEAD_DIM:(hh + 1) * HEAD_DIM]
                k = qkv[:, EMBED + hh * HEAD_DIM:EMBED + (hh + 1) * HEAD_DIM]
                v = qkv[:, 2 * EMBED + hh * HEAD_DIM:
                        2 * EMBED + (hh + 1) * HEAD_DIM]
                scores = lax.dot_general(
                    q, k, (((1,), (1,)), ((), ())),
                    preferred_element_type=jnp.float32) / t
                scores = jnp.where(diag, -1e9, scores)
                m = jnp.max(scores, axis=-1, keepdims=True)
                e = jnp.exp(scores - m)
                attn = e / jnp.sum(e, axis=-1, keepdims=True)
                o_heads.append(jnp.dot(attn, v,
                                       preferred_element_type=jnp.float32))
            o = jnp.concatenate(o_heads, axis=-1)
            x = x + jnp.dot(o, wo[blk],
                            preferred_element_type=jnp.float32) + bo[blk]
            h2 = _ln(x, ln2g[blk], ln2b[blk])
            h2 = jax.nn.gelu(jnp.dot(h2, w1[blk],
                                     preferred_element_type=jnp.float32)
                             + b1[blk])
            h2 = jnp.dot(h2, w2[blk],
                         preferred_element_type=jnp.float32) + b2[blk]
            x = x + h2
        out_ref[i] = jnp.concatenate(
            [x, jnp.zeros((SEQP - SEQ, EMBED), jnp.float32)], axis=0)
        return 0

    lax.fori_loop(0, BB, per_image, 0)


def _run_transformer(patches, p):
    whole = lambda shape: pl.BlockSpec(shape, lambda m: (0,) * len(shape))
    grid = BATCH // BB
    return pl.pallas_call(
        _transformer_body,
        grid=(grid,),
        in_specs=[
            pl.BlockSpec((BB, SEQP, EMBED), lambda m: (m, 0, 0)),
            whole((NUM_BLOCKS, EMBED)), whole((NUM_BLOCKS, EMBED)),
            whole((NUM_BLOCKS, EMBED, 3 * EMBED)),
            whole((NUM_BLOCKS, 3 * EMBED)),
            whole((NUM_BLOCKS, EMBED, EMBED)), whole((NUM_BLOCKS, EMBED)),
            whole((NUM_BLOCKS, 1)),
            whole((NUM_BLOCKS, EMBED)), whole((NUM_BLOCKS, EMBED)),
            whole((NUM_BLOCKS, EMBED, HIDDEN)), whole((NUM_BLOCKS, HIDDEN)),
            whole((NUM_BLOCKS, HIDDEN, EMBED)), whole((NUM_BLOCKS, EMBED)),
        ],
        out_specs=pl.BlockSpec((BB, SEQP, EMBED), lambda m: (m, 0, 0)),
        out_shape=jax.ShapeDtypeStruct((BATCH, SEQP, EMBED), jnp.float32),
    )(jnp.pad(patches, ((0, 0), (0, SEQP - SEQ), (0, 0))), *p)


BM = BATCH
BN = 256
KDIM = SEQ * EMBED
PW = (MAX_ANGLE - MIN_ANGLE) / IMAGE_SIZE
CX = IMAGE_SIZE // 2


def _proj_body(flat_ref, wout_ref, bout_ref, dist_ref, xg_ref, yg_ref, rg_ref,
               ks_ref, idx_ref):
    ks = jnp.dot(flat_ref[...], wout_ref[...],
                 preferred_element_type=jnp.float32) + bout_ref[...]
    ks_ref[...] = ks
    ea = ks * dist_ref[...]
    xg = xg_ref[...]
    yg = yg_ref[...]
    rg = rg_ref[...]
    xdef = ea * xg / rg
    ydef = ea * yg / rg
    bx = (xg - xdef) / PW
    by = (yg - ydef) / PW
    bx = jnp.clip(bx + CX, 0, IMAGE_SIZE - 1).astype(jnp.int32)
    by = jnp.clip(by + CX, 0, IMAGE_SIZE - 1).astype(jnp.int32)
    idx = bx * IMAGE_SIZE + by
    riota = lax.broadcasted_iota(jnp.int32, (BM, BN), 0)
    off = ((riota // IMGS_PER_WORKER) % SC_SUBCORES) * (2 * PIX)
    idx_ref[:, 0, :] = idx + off
    idx_ref[:, 1, :] = idx + off + PIX


def _run_projection(flat, wout, bout2, dist2, xg2, yg2, rg2):
    return pl.pallas_call(
        _proj_body,
        grid=(PIX // BN,),
        in_specs=[
            pl.BlockSpec((BM, KDIM), lambda n: (0, 0)),
            pl.BlockSpec((KDIM, BN), lambda n: (0, n)),
            pl.BlockSpec((1, BN), lambda n: (0, n)),
            pl.BlockSpec((BM, BN), lambda n: (0, n)),
            pl.BlockSpec((1, BN), lambda n: (0, n)),
            pl.BlockSpec((1, BN), lambda n: (0, n)),
            pl.BlockSpec((1, BN), lambda n: (0, n)),
        ],
        out_specs=[
            pl.BlockSpec((BM, BN), lambda n: (0, n)),
            pl.BlockSpec((BM, 2, BN), lambda n: (0, 0, n)),
        ],
        out_shape=[
            jax.ShapeDtypeStruct((BATCH, PIX), jnp.float32),
            jax.ShapeDtypeStruct((BATCH, 2, PIX), jnp.int32),
        ],
    )(flat, wout, bout2, dist2, xg2, yg2, rg2)


NVEC = PIX // 16
REGION = 2 * PIX
CHUNK = 128
NCHUNK = REGION // CHUNK


def _scatter_body(val_hbm, idx_hbm, out_hbm,
                  idx_v, val_v, acc_v, out_v, zero_v, shared):
    sidx = lax.axis_index("s")
    wid = lax.axis_index("c") * SC_SUBCORES + sidx
    ones16 = jnp.full((16,), 1.0, jnp.float32)
    zeros16 = jnp.zeros((16,), jnp.float32)

    def mkzero(i, _):
        zero_v[pl.ds(i * 16, 16)] = zeros16
        return 0

    lax.fori_loop(0, REGION // 16, mkzero, 0)
    region = pl.ds(sidx * REGION, REGION)

    def per_image(b, _):
        row = wid * IMGS_PER_WORKER + b
        pltpu.sync_copy(idx_hbm.at[row], idx_v)
        pltpu.sync_copy(val_hbm.at[row], val_v)
        pltpu.sync_copy(zero_v, shared.at[region])

        def chunk(j, _):
            pltpu.sync_copy(val_v.at[j], shared.at[idx_v.at[j]], add=True)
            return 0

        lax.fori_loop(0, NCHUNK, chunk, 0)
        pltpu.sync_copy(shared.at[region], acc_v)

        def fin(i, _):
            sl = pl.ds(i * 16, 16)
            s = acc_v[sl]
            c = acc_v[pl.ds(PIX + i * 16, 16)]
            nz = c != 0.0
            safe = jnp.where(nz, c, ones16)
            out_v[sl] = jnp.where(nz, s / safe, s)
            return 0

        lax.fori_loop(0, NVEC, fin, 0)
        pltpu.sync_copy(out_v, out_hbm.at[row])
        return 0

    lax.fori_loop(0, IMGS_PER_WORKER, per_image, 0)


def _run_scatter(val3, idx3):
    mesh = plsc.VectorSubcoreMesh(core_axis_name="c", subcore_axis_name="s",
                                  num_cores=SC_CORES,
                                  num_subcores=SC_SUBCORES)
    f = pl.kernel(
        _scatter_body,
        out_type=jax.ShapeDtypeStruct((BATCH, PIX), jnp.float32),
        mesh=mesh,
        scratch_types=[
            pltpu.VMEM((NCHUNK, CHUNK), jnp.int32),
            pltpu.VMEM((NCHUNK, CHUNK), jnp.float32),
            pltpu.VMEM((REGION,), jnp.float32),
            pltpu.VMEM((PIX,), jnp.float32),
            pltpu.VMEM((REGION,), jnp.float32),
            pltpu.VMEM_SHARED((SC_SUBCORES * REGION,), jnp.float32),
        ],
    )
    return f(val3, idx3)


def _lens_grids():
    pw = (MAX_ANGLE - MIN_ANGLE) / IMAGE_SIZE
    cx = IMAGE_SIZE // 2
    rx = jnp.arange(-(cx - 1), IMAGE_SIZE - (cx - 1))
    x, y = jnp.meshgrid(rx, rx, indexing='ij')
    x = x * pw
    y = y * pw
    r = jnp.sqrt(x ** 2 + y ** 2)
    r = jnp.where(r == 0, 1.0, r)
    return (x.reshape(1, PIX).astype(jnp.float32),
            y.reshape(1, PIX).astype(jnp.float32),
            r.reshape(1, PIX).astype(jnp.float32))


@jax.jit
def kernel(input_images, patches, distortion, params):
    blocks = params['blocks']
    names = ['ln1_g', 'ln1_b', 'Wqkv', 'bqkv', 'Wo', 'bo', 'temp',
             'ln2_g', 'ln2_b', 'W1', 'b1', 'W2', 'b2']
    stacked = []
    for nm in names:
        arrs = [jnp.asarray(b[nm], jnp.float32) for b in blocks]
        if nm == 'temp':
            arrs = [a.reshape(1) for a in arrs]
        stacked.append(jnp.stack(arrs, axis=0))

    x = _run_transformer(patches, stacked)[:, :SEQ, :]
    flat = x.reshape(BATCH, SEQ * EMBED)

    xg2, yg2, rg2 = _lens_grids()
    dist2 = distortion.reshape(BATCH, PIX)
    bout2 = params['bout'].reshape(1, PIX)
    ks2, idx2 = _run_projection(flat, params['Wout'], bout2, dist2,
                                xg2, yg2, rg2)
    val3 = jnp.concatenate(
        [input_images.reshape(BATCH, PIX),
         jnp.ones((BATCH, PIX), jnp.float32)], axis=1).reshape(
             BATCH, NCHUNK, CHUNK)
    idx3 = idx2.reshape(BATCH, NCHUNK, CHUNK)
    src2 = _run_scatter(val3, idx3)

    return (ks2.reshape(BATCH, IMAGE_SIZE, IMAGE_SIZE),
            src2.reshape(BATCH, IMAGE_SIZE, IMAGE_SIZE))
